# baseline (speedup 1.0000x reference)
import jax
import jax.numpy as jnp
from jax.experimental import pallas as pl
from jax.experimental.pallas import tpu as pltpu

_BI = 512
_BK = 512


def _pass1(adj_ref, seq_ref, w_ref, bias_ref, out_ref, sf_ref, acc_ref):
    i = pl.program_id(0)
    k = pl.program_id(1)
    nk = pl.num_programs(1)

    @pl.when(i == 0)
    def _compute_sf_block():
        sfk = jax.lax.dot_general(
            seq_ref[pl.ds(k * _BK, _BK), :].astype(jnp.bfloat16),
            w_ref[...].astype(jnp.bfloat16),
            (((1,), (1,)), ((), ())),
            preferred_element_type=jnp.float32,
        )
        sf_ref[pl.ds(k * _BK, _BK), :] = sfk.astype(jnp.bfloat16)

    @pl.when(k == 0)
    def _init():
        acc_ref[...] = jnp.zeros_like(acc_ref)

    acc_ref[...] += jax.lax.dot_general(
        adj_ref[...].astype(jnp.bfloat16),
        sf_ref[pl.ds(k * _BK, _BK), :],
        (((1,), (0,)), ((), ())),
        preferred_element_type=jnp.float32,
    )

    @pl.when(k == nk - 1)
    def _epilogue():
        out_ref[...] = (acc_ref[...] + bias_ref[...]).astype(jnp.bfloat16)


def _pass2(adj_ref, h_ref, bias_ref, a_ref, out_ref, acc_ref):
    k = pl.program_id(1)
    nk = pl.num_programs(1)

    @pl.when(k == 0)
    def _init():
        acc_ref[...] = jnp.zeros_like(acc_ref)

    acc_ref[...] += jax.lax.dot_general(
        adj_ref[...].astype(jnp.bfloat16),
        h_ref[pl.ds(k * _BK, _BK), :],
        (((1,), (0,)), ((), ())),
        preferred_element_type=jnp.float32,
    )

    @pl.when(k == nk - 1)
    def _epilogue():
        o = acc_ref[...] + bias_ref[...]
        out_ref[...] = jnp.where(o > 0, o, a_ref[0, 0] * o)


def kernel(seq, adj, du, W, bias, prelu_a):
    del du
    (b, n, f_in) = seq.shape
    f_out = W.shape[0]
    seq2 = seq.reshape(n, f_in)
    adj2 = adj.reshape(n, n)
    bias2 = bias.reshape(1, f_out)
    a2 = jnp.reshape(prelu_a, (1, 1)).astype(jnp.float32)

    ni = n // _BI
    nk = n // _BK

    h = pl.pallas_call(
        _pass1,
        grid=(ni, nk),
        in_specs=[
            pl.BlockSpec((_BI, _BK), lambda i, k: (i, k)),
            pl.BlockSpec((n, f_in), lambda i, k: (0, 0)),
            pl.BlockSpec((f_out, f_in), lambda i, k: (0, 0)),
            pl.BlockSpec((1, f_out), lambda i, k: (0, 0)),
        ],
        out_specs=pl.BlockSpec((_BI, f_out), lambda i, k: (i, 0)),
        out_shape=jax.ShapeDtypeStruct((n, f_out), jnp.bfloat16),
        scratch_shapes=[
            pltpu.VMEM((n, f_out), jnp.bfloat16),
            pltpu.VMEM((_BI, f_out), jnp.float32),
        ],
    )(adj2, seq2, W, bias2)

    out = pl.pallas_call(
        _pass2,
        grid=(ni, nk),
        in_specs=[
            pl.BlockSpec((_BI, _BK), lambda i, k: (i, k)),
            pl.BlockSpec((n, f_out), lambda i, k: (0, 0)),
            pl.BlockSpec((1, f_out), lambda i, k: (0, 0)),
            pl.BlockSpec((1, 1), lambda i, k: (0, 0)),
        ],
        out_specs=pl.BlockSpec((_BI, f_out), lambda i, k: (i, 0)),
        out_shape=jax.ShapeDtypeStruct((n, f_out), jnp.float32),
        scratch_shapes=[
            pltpu.VMEM((_BI, f_out), jnp.float32),
        ],
    )(adj2, h, bias2, a2)

    return out.reshape(b, n, f_out)
